# baseline (speedup 1.0000x reference)
import functools
import jax
import jax.numpy as jnp
from jax import lax
from jax.experimental import pallas as pl
from jax.experimental.pallas import tpu as pltpu
from jax.experimental.pallas import tpu_sc as plsc


_GATHER_G = 8
_GATHER_LANES = 128


def _gcd(a, b):
    while b:
        a, b = b, a % b
    return a


def _sc_gather(table, idx_pad2, groups_per_worker):
    v, d = table.shape
    r = idx_pad2.shape[0]
    nw = 32
    rows_buf = _GATHER_G * _GATHER_LANES

    mesh = plsc.VectorSubcoreMesh(core_axis_name="c", subcore_axis_name="s")

    @functools.partial(
        pl.kernel,
        out_type=jax.ShapeDtypeStruct((r * _GATHER_LANES, d), jnp.float32),
        mesh=mesh,
        compiler_params=pltpu.CompilerParams(use_tc_tiling_on_sc=False),
        scratch_types=[
            pltpu.VMEM((_GATHER_G, _GATHER_LANES), jnp.int32),
            pltpu.VMEM((rows_buf, d), jnp.float32),
            pltpu.SemaphoreType.DMA,
        ],
    )
    def k(table_hbm, idx_hbm, out_hbm, idx_v, rows_v, sem):
        nc = 2
        wid = lax.axis_index("s") * nc + lax.axis_index("c")
        base = wid * (groups_per_worker * _GATHER_G)

        def body(g, carry):
            row0 = base + g * _GATHER_G
            pltpu.sync_copy(idx_hbm.at[pl.ds(row0, _GATHER_G)], idx_v)
            copies = []
            for j in range(_GATHER_G):
                copies.append(pltpu.async_copy(
                    table_hbm.at[idx_v.at[j]],
                    rows_v.at[pl.ds(j * _GATHER_LANES, _GATHER_LANES)],
                    sem))
            for c in copies:
                c.wait()
            pltpu.sync_copy(rows_v,
                            out_hbm.at[pl.ds(row0 * _GATHER_LANES, rows_buf)])
            return carry

        lax.fori_loop(0, groups_per_worker, body, None)

    return k(table, idx_pad2)


def _enc_body(x_ref, w_ref, o_ref):
    o_ref[...] = jnp.maximum(x_ref[...] @ w_ref[...], 0.0)


def _encode(features, W_pw, bm):
    n, k = features.shape
    d = W_pw.shape[1]
    return pl.pallas_call(
        _enc_body,
        grid=(n // bm,),
        in_specs=[
            pl.BlockSpec((bm, k), lambda i: (i, 0)),
            pl.BlockSpec((k, d), lambda i: (0, 0)),
        ],
        out_specs=pl.BlockSpec((bm, d), lambda i: (i, 0)),
        out_shape=jax.ShapeDtypeStruct((n, d), jnp.float32),
    )(features, W_pw)


def _prop_body(a_ref, x_ref, o_ref):
    o_ref[...] = a_ref[...] @ x_ref[...]


def _prop2_body(a_ref, x_ref, e_ref, t_ref, o_ref):
    o_ref[...] = (a_ref[...] @ x_ref[...] + e_ref[...] + t_ref[...]) * (1.0 / 3.0)


def _propagate(adj, emb0, bm):
    n, d = emb0.shape
    t1 = pl.pallas_call(
        _prop_body,
        grid=(n // bm,),
        in_specs=[
            pl.BlockSpec((bm, n), lambda i: (i, 0)),
            pl.BlockSpec((n, d), lambda i: (0, 0)),
        ],
        out_specs=pl.BlockSpec((bm, d), lambda i: (i, 0)),
        out_shape=jax.ShapeDtypeStruct((n, d), jnp.float32),
    )(adj, emb0)
    gnn = pl.pallas_call(
        _prop2_body,
        grid=(n // bm,),
        in_specs=[
            pl.BlockSpec((bm, n), lambda i: (i, 0)),
            pl.BlockSpec((n, d), lambda i: (0, 0)),
            pl.BlockSpec((bm, d), lambda i: (i, 0)),
            pl.BlockSpec((bm, d), lambda i: (i, 0)),
        ],
        out_specs=pl.BlockSpec((bm, d), lambda i: (i, 0)),
        out_shape=jax.ShapeDtypeStruct((n, d), jnp.float32),
    )(adj, t1, emb0, t1)
    return gnn


def _lstm_body(pe_ref, len_ref, wih_ref, whh_ref, b_ref, wo_ref, bo_ref,
               o_ref, *, nl, h_dim):
    x = pe_ref[...]
    bp = x.shape[0]
    b = b_ref[...]
    wcat = jnp.concatenate([wih_ref[...], whh_ref[...]], axis=0)
    idx = jnp.clip(len_ref[...] - 1, 0, nl - 1)
    h = jnp.zeros((bp, h_dim), jnp.float32)
    c = jnp.zeros((bp, h_dim), jnp.float32)
    h_last = jnp.zeros((bp, h_dim), jnp.float32)
    d = x.shape[1] // nl
    for l in range(nl):
        x_t = x[:, l * d:(l + 1) * d]
        z = jnp.concatenate([x_t, h], axis=1) @ wcat + b
        i_g = jax.nn.sigmoid(z[:, :h_dim])
        f_g = jax.nn.sigmoid(z[:, h_dim:2 * h_dim])
        g_g = jnp.tanh(z[:, 2 * h_dim:3 * h_dim])
        o_g = jax.nn.sigmoid(z[:, 3 * h_dim:])
        c = f_g * c + i_g * g_g
        h = o_g * jnp.tanh(c)
        h_last = jnp.where(idx == l, h, h_last)
    pw = jax.nn.sigmoid(h_last @ wo_ref[...] + bo_ref[0, 0])
    o_ref[...] = pw


def _lstm_pw(path_emb, lengths, W_ih, W_hh, b_ih, b_hh, w_out, b_out, bp):
    p, ld = path_emb.shape
    h_dim = W_hh.shape[1]
    nl = ld // (W_ih.shape[1])
    wih = W_ih.T
    whh = W_hh.T
    b = (b_ih + b_hh).reshape(1, -1)
    wo = w_out.reshape(-1, 1)
    bo = b_out.reshape(1, 1)
    lengths2 = lengths.reshape(p, 1)
    pw2 = pl.pallas_call(
        functools.partial(_lstm_body, nl=nl, h_dim=h_dim),
        grid=(p // bp,),
        in_specs=[
            pl.BlockSpec((bp, ld), lambda i: (i, 0)),
            pl.BlockSpec((bp, 1), lambda i: (i, 0)),
            pl.BlockSpec(wih.shape, lambda i: (0, 0)),
            pl.BlockSpec(whh.shape, lambda i: (0, 0)),
            pl.BlockSpec(b.shape, lambda i: (0, 0)),
            pl.BlockSpec(wo.shape, lambda i: (0, 0)),
            pl.BlockSpec(bo.shape, lambda i: (0, 0)),
        ],
        out_specs=pl.BlockSpec((bp, 1), lambda i: (i, 0)),
        out_shape=jax.ShapeDtypeStruct((p, 1), jnp.float32),
    )(path_emb, lengths2, wih, whh, b, wo, bo)
    return pw2.reshape(p)


_SC_PARAMS = pltpu.CompilerParams(use_tc_tiling_on_sc=False)
_NW = 32


def _sc_mesh():
    return plsc.VectorSubcoreMesh(core_axis_name="c", subcore_axis_name="s")


def _iota16():
    return lax.iota(jnp.int32, 16)


def _sc_scatter_ids(cell2, nn_cells):
    r = cell2.shape[0]
    wr = r // _NW

    @functools.partial(
        pl.kernel,
        out_type=jax.ShapeDtypeStruct((nn_cells,), jnp.int32),
        mesh=_sc_mesh(),
        compiler_params=_SC_PARAMS,
        scratch_types=[
            pltpu.VMEM((wr, 128), jnp.int32),
            pltpu.VMEM((8, 128), jnp.int32),
            pltpu.SemaphoreType.DMA,
        ],
    )
    def k(cell_hbm, out_hbm, cidx_v, vals_v, sem):
        w = lax.axis_index("s") * 2 + lax.axis_index("c")
        base = w * wr
        pltpu.sync_copy(cell_hbm.at[pl.ds(base, wr)], cidx_v)

        def grp(g, carry):
            for j in range(8):
                rowbase = (base + g * 8 + j) * 128
                for q in range(8):
                    vals_v[j, pl.ds(q * 16, 16)] = rowbase + q * 16 + _iota16()
            cs = []
            for j in range(8):
                cs.append(pltpu.async_copy(
                    vals_v.at[j], out_hbm.at[cidx_v.at[g * 8 + j]], sem))
            for c in cs:
                c.wait()
            return carry

        lax.fori_loop(0, wr // 8, grp, 0)

    return k(cell2)


def _sc_winner_extra(cell2, pw2, cellbuf):
    r = cell2.shape[0]
    wr = r // _NW
    p2 = r * 128
    sl = p2 // 16

    @functools.partial(
        pl.kernel,
        out_type=(jax.ShapeDtypeStruct((r, 128), jnp.int32),
                  jax.ShapeDtypeStruct((2, p2), jnp.float32)),
        mesh=_sc_mesh(),
        compiler_params=_SC_PARAMS,
        scratch_types=[
            pltpu.VMEM((wr, 128), jnp.int32),
            pltpu.VMEM((wr, 128), jnp.int32),
            pltpu.VMEM((wr, 128), jnp.float32),
            pltpu.VMEM((wr, 128), jnp.float32),
            pltpu.VMEM_SHARED((p2,), jnp.float32),
            pltpu.SemaphoreType.DMA,
        ],
    )
    def k(cell_hbm, pw_hbm, cb_hbm, z_hbm, wid_out, extra_out,
          cidx_v, wid_v, pw_v, lv_v, spart, sem):
        c = lax.axis_index("c")
        s = lax.axis_index("s")
        w = s * 2 + c
        pltpu.sync_copy(z_hbm.at[pl.ds(s * sl, sl)],
                        spart.at[pl.ds(s * sl, sl)])
        plsc.subcore_barrier()
        base = w * wr
        pltpu.sync_copy(cell_hbm.at[pl.ds(base, wr)], cidx_v)
        pltpu.sync_copy(pw_hbm.at[pl.ds(base, wr)], pw_v)

        def ggrp(g, carry):
            cs = []
            for j in range(8):
                row = g * 8 + j
                cs.append(pltpu.async_copy(
                    cb_hbm.at[cidx_v.at[row]], wid_v.at[row], sem))
            for cpy in cs:
                cpy.wait()
            return carry

        lax.fori_loop(0, wr // 8, ggrp, 0)

        def comp(j, carry):
            rowbase = (base + j) * 128
            for q in range(8):
                idv = rowbase + q * 16 + _iota16()
                wv = wid_v[j, pl.ds(q * 16, 16)]
                pv = pw_v[j, pl.ds(q * 16, 16)]
                lv_v[j, pl.ds(q * 16, 16)] = jnp.where(
                    wv == idv, jnp.zeros((16,), jnp.float32), pv)
            return carry

        lax.fori_loop(0, wr, comp, 0)

        def sgrp(j, carry):
            pltpu.sync_copy(lv_v.at[j], spart.at[wid_v.at[j]], add=True)
            return carry

        lax.fori_loop(0, wr, sgrp, 0)
        plsc.subcore_barrier()
        pltpu.sync_copy(spart.at[pl.ds(s * sl, sl)],
                        extra_out.at[c, pl.ds(s * sl, sl)])
        pltpu.sync_copy(wid_v, wid_out.at[pl.ds(base, wr)])

    return k(cell2, pw2, cellbuf, jnp.zeros((p2,), jnp.float32))


def _sc_stats(wid2, row2, col2, combined, n):
    r = wid2.shape[0]
    wr = r // _NW
    na = ((n + 127) // 128) * 128
    sl = na // 16

    @functools.partial(
        pl.kernel,
        out_type=(jax.ShapeDtypeStruct((r, 128), jnp.float32),
                  jax.ShapeDtypeStruct((2, na), jnp.float32),
                  jax.ShapeDtypeStruct((2, na), jnp.float32)),
        mesh=_sc_mesh(),
        compiler_params=_SC_PARAMS,
        scratch_types=[
            pltpu.VMEM((wr, 128), jnp.int32),
            pltpu.VMEM((wr, 128), jnp.int32),
            pltpu.VMEM((wr, 128), jnp.int32),
            pltpu.VMEM((wr, 128), jnp.float32),
            pltpu.VMEM((wr, 128), jnp.float32),
            pltpu.VMEM((wr, 128), jnp.float32),
            pltpu.VMEM_SHARED((na,), jnp.float32),
            pltpu.VMEM_SHARED((na,), jnp.float32),
            pltpu.SemaphoreType.DMA,
        ],
    )
    def k(wid_hbm, row_hbm, col_hbm, comb_hbm, z_hbm,
          ex_out, zsum_out, dsum_out,
          wid_v, row_v, col_v, t_v, zv_v, dv_v, zpart, dpart, sem):
        c = lax.axis_index("c")
        s = lax.axis_index("s")
        w = s * 2 + c
        pltpu.sync_copy(z_hbm.at[pl.ds(s * sl, sl)],
                        zpart.at[pl.ds(s * sl, sl)])
        pltpu.sync_copy(z_hbm.at[pl.ds(s * sl, sl)],
                        dpart.at[pl.ds(s * sl, sl)])
        plsc.subcore_barrier()
        base = w * wr
        pltpu.sync_copy(wid_hbm.at[pl.ds(base, wr)], wid_v)
        pltpu.sync_copy(row_hbm.at[pl.ds(base, wr)], row_v)
        pltpu.sync_copy(col_hbm.at[pl.ds(base, wr)], col_v)

        def ggrp(g, carry):
            cs = []
            for j in range(8):
                row = g * 8 + j
                cs.append(pltpu.async_copy(
                    comb_hbm.at[wid_v.at[row]], t_v.at[row], sem))
            for cpy in cs:
                cpy.wait()
            return carry

        lax.fori_loop(0, wr // 8, ggrp, 0)

        def comp(j, carry):
            rowbase = (base + j) * 128
            for q in range(8):
                sli = pl.ds(q * 16, 16)
                idv = rowbase + q * 16 + _iota16()
                wv = wid_v[j, sli]
                rv = row_v[j, sli]
                cv = col_v[j, sli]
                tv = t_v[j, sli]
                ex = jnp.exp(tv)
                winner = wv == idv
                offd = rv != cv
                zero = jnp.zeros((16,), jnp.float32)
                exw = jnp.where(winner, ex, zero)
                tw = jnp.where(winner, tv, zero)
                zv_v[j, sli] = jnp.where(offd, exw, zero)
                dv_v[j, sli] = jnp.where(offd, zero, tw)
                t_v[j, sli] = ex
            return carry

        lax.fori_loop(0, wr, comp, 0)

        def sgrp(j, carry):
            pltpu.sync_copy(zv_v.at[j], zpart.at[row_v.at[j]], add=True)
            pltpu.sync_copy(dv_v.at[j], dpart.at[row_v.at[j]], add=True)
            return carry

        lax.fori_loop(0, wr, sgrp, 0)
        plsc.subcore_barrier()
        pltpu.sync_copy(zpart.at[pl.ds(s * sl, sl)],
                        zsum_out.at[c, pl.ds(s * sl, sl)])
        pltpu.sync_copy(dpart.at[pl.ds(s * sl, sl)],
                        dsum_out.at[c, pl.ds(s * sl, sl)])
        pltpu.sync_copy(t_v, ex_out.at[pl.ds(base, wr)])

    return k(wid2, row2, col2, combined, jnp.zeros((na,), jnp.float32))


def _sc_scatter_vals(cellF, rowF, colF, exF, widF, Z, dv, pw_ref):
    r = cellF.shape[0]
    wr = r // _NW
    ng = wr // 8
    rem = wr - ng * 8

    @functools.partial(
        pl.kernel,
        out_type=jax.ShapeDtypeStruct((r, 128), jnp.float32),
        mesh=_sc_mesh(),
        compiler_params=_SC_PARAMS,
        scratch_types=[
            pltpu.VMEM((wr, 128), jnp.int32),
            pltpu.VMEM((wr, 128), jnp.int32),
            pltpu.VMEM((wr, 128), jnp.int32),
            pltpu.VMEM((wr, 128), jnp.int32),
            pltpu.VMEM((wr, 128), jnp.float32),
            pltpu.VMEM((wr, 128), jnp.float32),
            pltpu.VMEM((wr, 128), jnp.float32),
            pltpu.VMEM((wr, 128), jnp.float32),
            pltpu.VMEM((wr, 128), jnp.float32),
            pltpu.SemaphoreType.DMA,
        ],
    )
    def k(cell_hbm, row_hbm, col_hbm, ex_hbm, wid_hbm, zt_hbm, dvt_hbm,
          pw_adj_ref, vsc_out,
          cidx_v, row_v, col_v, wid_v, ex_v, zr_v, dvr_v, v_v, vsc_v, sem):
        w = lax.axis_index("s") * 2 + lax.axis_index("c")
        base = w * wr
        pltpu.sync_copy(cell_hbm.at[pl.ds(base, wr)], cidx_v)
        pltpu.sync_copy(row_hbm.at[pl.ds(base, wr)], row_v)
        pltpu.sync_copy(col_hbm.at[pl.ds(base, wr)], col_v)
        pltpu.sync_copy(ex_hbm.at[pl.ds(base, wr)], ex_v)
        pltpu.sync_copy(wid_hbm.at[pl.ds(base, wr)], wid_v)

        def ggrp(g, carry):
            cs = []
            for j in range(8):
                rw = g * 8 + j
                cs.append(pltpu.async_copy(
                    zt_hbm.at[row_v.at[rw]], zr_v.at[rw], sem))
                cs.append(pltpu.async_copy(
                    dvt_hbm.at[row_v.at[rw]], dvr_v.at[rw], sem))
            for cpy in cs:
                cpy.wait()
            return carry

        lax.fori_loop(0, ng, ggrp, 0)
        if rem:
            cs = []
            for j in range(rem):
                rw = ng * 8 + j
                cs.append(pltpu.async_copy(
                    zt_hbm.at[row_v.at[rw]], zr_v.at[rw], sem))
                cs.append(pltpu.async_copy(
                    dvt_hbm.at[row_v.at[rw]], dvr_v.at[rw], sem))
            for cpy in cs:
                cpy.wait()

        def comp(j, carry):
            rowbase = (base + j) * 128
            for q in range(8):
                sli = pl.ds(q * 16, 16)
                idv = rowbase + q * 16 + _iota16()
                rv = row_v[j, sli]
                cv = col_v[j, sli]
                wv = wid_v[j, sli]
                ex = ex_v[j, sli]
                diag = rv == cv
                zero = jnp.zeros((16,), jnp.float32)
                v = jnp.where(diag, dvr_v[j, sli], ex / zr_v[j, sli])
                v_v[j, sli] = v
                vw = jnp.where(wv == idv, v, zero)
                vsc_v[j, sli] = jnp.where(diag, zero, vw)
            return carry

        lax.fori_loop(0, wr, comp, 0)

        def sgrp(g, carry):
            cs = []
            for j in range(8):
                rw = g * 8 + j
                cs.append(pltpu.async_copy(
                    v_v.at[rw], pw_adj_ref.at[cidx_v.at[rw]], sem))
            for cpy in cs:
                cpy.wait()
            return carry

        lax.fori_loop(0, ng, sgrp, 0)
        if rem:
            cs = []
            for j in range(rem):
                rw = ng * 8 + j
                cs.append(pltpu.async_copy(
                    v_v.at[rw], pw_adj_ref.at[cidx_v.at[rw]], sem))
            for cpy in cs:
                cpy.wait()
        pltpu.sync_copy(vsc_v, vsc_out.at[pl.ds(base, wr)])

    return k(cellF, rowF, colF, exF, widF, Z, dv, pw_ref)


def _sc_spmm(vexp, row2, col2, gnn):
    r = row2.shape[0]
    wr = r // _NW
    n, d = gnn.shape
    na = ((n + 127) // 128) * 128
    sl = na // 16

    @functools.partial(
        pl.kernel,
        out_type=jax.ShapeDtypeStruct((2, na, d), jnp.float32),
        mesh=_sc_mesh(),
        compiler_params=_SC_PARAMS,
        scratch_types=[
            pltpu.VMEM((wr, 128), jnp.int32),
            pltpu.VMEM((wr, 128), jnp.int32),
            pltpu.VMEM((128, 64), jnp.float32),
            pltpu.VMEM((128, 64), jnp.float32),
            pltpu.VMEM_SHARED((na, 64), jnp.float32),
            pltpu.SemaphoreType.DMA,
        ],
    )
    def k(vexp_hbm, row_hbm, col_hbm, gnn_hbm, z_hbm,
          pe_out, row_v, col_v, rows_v, vex_v, pepart, sem):
        c = lax.axis_index("c")
        s = lax.axis_index("s")
        w = s * 2 + c
        pltpu.sync_copy(z_hbm.at[pl.ds(s * sl, sl)],
                        pepart.at[pl.ds(s * sl, sl)])
        plsc.subcore_barrier()
        base = w * wr
        pltpu.sync_copy(row_hbm.at[pl.ds(base, wr)], row_v)
        pltpu.sync_copy(col_hbm.at[pl.ds(base, wr)], col_v)

        def body(j, carry):
            cg = pltpu.async_copy(gnn_hbm.at[col_v.at[j]], rows_v, sem)
            cv = pltpu.async_copy(
                vexp_hbm.at[pl.ds((base + j) * 128, 128)], vex_v, sem)
            cg.wait()
            cv.wait()
            for kk in range(128):
                for qq in range(4):
                    sli = pl.ds(qq * 16, 16)
                    rows_v[kk, sli] = rows_v[kk, sli] * vex_v[kk, sli]
            pltpu.sync_copy(rows_v, pepart.at[row_v.at[j]], add=True)
            return carry

        lax.fori_loop(0, wr, body, 0)
        plsc.subcore_barrier()
        pltpu.sync_copy(pepart.at[pl.ds(s * sl, sl)],
                        pe_out.at[c, pl.ds(s * sl, sl)])

    return k(vexp, row2, col2, gnn, jnp.zeros((na, d), jnp.float32))


def _comb_body(pw_ref, e0_ref, e1_ref, o_ref):
    o_ref[...] = pw_ref[...] + e0_ref[...] + e1_ref[...]


def _combine(pw2, e0, e1):
    r = pw2.shape[0]
    return pl.pallas_call(
        _comb_body,
        grid=(1,),
        in_specs=[pl.BlockSpec((r, 128), lambda i: (0, 0))] * 3,
        out_specs=pl.BlockSpec((r, 128), lambda i: (0, 0)),
        out_shape=jax.ShapeDtypeStruct((r, 128), jnp.float32),
    )(pw2, e0, e1)


def _zstats_body(z0_ref, z1_ref, d0_ref, d1_ref, z_ref, dv_ref):
    de = jnp.exp(1.0 + d0_ref[...] + d1_ref[...])
    zt = z0_ref[...] + z1_ref[...] + de
    z_ref[...] = zt
    dv_ref[...] = de / zt


def _zstats(zsum, dsum, n):
    return pl.pallas_call(
        _zstats_body,
        grid=(1,),
        in_specs=[pl.BlockSpec((1, n), lambda i: (0, 0))] * 4,
        out_specs=[pl.BlockSpec((1, n), lambda i: (0, 0))] * 2,
        out_shape=[jax.ShapeDtypeStruct((1, n), jnp.float32)] * 2,
    )(zsum[0:1], zsum[1:2], dsum[0:1], dsum[1:2])


def _zfill_body(o_ref):
    o_ref[...] = jnp.zeros_like(o_ref)


def _zero_dense(n, bm):
    return pl.pallas_call(
        _zfill_body,
        grid=(n // bm,),
        in_specs=[],
        out_specs=pl.BlockSpec((bm, n), lambda i: (i, 0)),
        out_shape=jax.ShapeDtypeStruct((n, n), jnp.float32),
    )()


def _smax_body(a_ref, g_ref, o_ref, e_ref):
    a = a_ref[...]
    aw = jnp.where(a > 0.0, a, jnp.float32(-9e15))
    m = jnp.max(aw, axis=1, keepdims=True)
    ex = jnp.exp(aw - m)
    s = jnp.sum(ex, axis=1, keepdims=True)
    p = ex / s
    o_ref[...] = p
    e_ref[...] = p @ g_ref[...]


def _softmax_spmm(A, gnn, bm):
    n = A.shape[0]
    d = gnn.shape[1]
    return pl.pallas_call(
        _smax_body,
        grid=(n // bm,),
        in_specs=[
            pl.BlockSpec((bm, n), lambda i: (i, 0)),
            pl.BlockSpec((n, d), lambda i: (0, 0)),
        ],
        out_specs=[
            pl.BlockSpec((bm, n), lambda i: (i, 0)),
            pl.BlockSpec((bm, d), lambda i: (i, 0)),
        ],
        out_shape=[
            jax.ShapeDtypeStruct((n, n), jnp.float32),
            jax.ShapeDtypeStruct((n, d), jnp.float32),
        ],
    )(A, gnn)


def _head_body(g_ref, p0_ref, p1_ref, dvv_ref, w1_ref, b1_ref, w2_ref,
               b2_ref, o_ref, *, lam):
    g = g_ref[...]
    pe = p0_ref[...] + p1_ref[...] + dvv_ref[...] * g
    e = jnp.concatenate([g, lam * pe], axis=1)
    h = jnp.maximum(e @ w1_ref[...] + b1_ref[...], 0.0)
    lg = h @ w2_ref[...] + b2_ref[...]
    m = jnp.max(lg, axis=1, keepdims=True)
    lse = m + jnp.log(jnp.sum(jnp.exp(lg - m), axis=1, keepdims=True))
    o_ref[...] = lg - lse


def _head(gnn, pe0, pe1, dvv, W1, b1, W2, b2, lam, bm):
    n, d = gnn.shape
    nh = W1.shape[1]
    nc = W2.shape[1]
    return pl.pallas_call(
        functools.partial(_head_body, lam=lam),
        grid=(n // bm,),
        in_specs=[
            pl.BlockSpec((bm, d), lambda i: (i, 0)),
            pl.BlockSpec((bm, d), lambda i: (i, 0)),
            pl.BlockSpec((bm, d), lambda i: (i, 0)),
            pl.BlockSpec((bm, 1), lambda i: (i, 0)),
            pl.BlockSpec(W1.shape, lambda i: (0, 0)),
            pl.BlockSpec((1, nh), lambda i: (0, 0)),
            pl.BlockSpec(W2.shape, lambda i: (0, 0)),
            pl.BlockSpec((1, nc), lambda i: (0, 0)),
        ],
        out_specs=pl.BlockSpec((bm, nc), lambda i: (i, 0)),
        out_shape=jax.ShapeDtypeStruct((n, nc), jnp.float32),
    )(gnn, pe0, pe1, dvv.reshape(n, 1), W1, b1.reshape(1, -1),
      W2, b2.reshape(1, -1))


def kernel(features, adj, pairs, sub_paths, sub_path_length, W_pw, W_ih, W_hh,
           b_ih, b_hh, w_out, b_out, W1, b1, W2, b2):
    n = features.shape[0]
    d = W_pw.shape[1]
    p, l = sub_paths.shape

    bm_enc = 2000 if n % 2000 == 0 else n
    emb0 = _encode(features, W_pw, bm_enc)

    bm = 1000 if n % 1000 == 0 else n
    bmp = 200 if n % 200 == 0 else n
    gnn = _propagate(adj, emb0, bmp)

    flat_idx = sub_paths.reshape(-1).astype(jnp.int32)
    unit = 32 * _GATHER_G * _GATHER_LANES
    flat_unit = (unit // _gcd(unit, l)) * l
    npad = ((p * l + flat_unit - 1) // flat_unit) * flat_unit
    idx_pad = jnp.pad(flat_idx, (0, npad - p * l))
    p_pad = npad // l
    lengths = jnp.pad(sub_path_length.astype(jnp.int32), (0, p_pad - p))
    nunits = npad // flat_unit
    nchunks = 1
    for cand in (5, 2):
        if nunits % cand == 0:
            nchunks = cand
            break
    cflat = npad // nchunks
    cpaths = p_pad // nchunks
    bp = 2048 if cpaths % 2048 == 0 else (2000 if cpaths % 2000 == 0 else cpaths)
    pw_parts = []
    for ci in range(nchunks):
        idx2 = idx_pad[ci * cflat:(ci + 1) * cflat].reshape(-1, _GATHER_LANES)
        rows = _sc_gather(gnn, idx2, cflat // unit)
        pe = rows.reshape(cpaths, l * d)
        ln = lengths[ci * cpaths:(ci + 1) * cpaths]
        pw_parts.append(_lstm_pw(pe, ln, W_ih, W_hh, b_ih, b_hh,
                                 w_out, b_out, bp))
    pw = jnp.concatenate(pw_parts)[:p]

    rowi = pairs[:, 0].astype(jnp.int32)
    coli = pairs[:, 1].astype(jnp.int32)
    celli = rowi * n + coli
    rows_p = ((p + 128 * 256 - 1) // (128 * 256)) * 256
    p2 = rows_p * 128
    pad_p = p2 - p
    cell2 = jnp.pad(celli, (0, pad_p)).reshape(rows_p, 128)
    row2 = jnp.pad(rowi, (0, pad_p)).reshape(rows_p, 128)
    col2 = jnp.pad(coli, (0, pad_p)).reshape(rows_p, 128)
    pw2 = jnp.pad(pw, (0, pad_p)).reshape(rows_p, 128)

    cellbuf = _sc_scatter_ids(cell2, n * n)
    wid2, extra = _sc_winner_extra(cell2, pw2, cellbuf)
    comb2 = _combine(pw2, extra[0].reshape(rows_p, 128),
                     extra[1].reshape(rows_p, 128))
    ex2, zsum, dsum = _sc_stats(wid2, row2, col2, comb2.reshape(p2), n)
    Zt, dvt = _zstats(zsum[:, :n], dsum[:, :n], n)
    Zt = Zt.reshape(n)
    dvt = dvt.reshape(n)

    rows_f = ((p2 + n + 128 * 256 - 1) // (128 * 256)) * 256
    pf = rows_f * 128
    pad_f = pf - p2 - n
    diag_idx = jnp.arange(n, dtype=jnp.int32)
    cellF = jnp.concatenate(
        [cell2.reshape(p2), diag_idx * (n + 1),
         jnp.zeros((pad_f,), jnp.int32)]).reshape(rows_f, 128)
    rowF = jnp.concatenate(
        [row2.reshape(p2), diag_idx,
         jnp.zeros((pad_f,), jnp.int32)]).reshape(rows_f, 128)
    colF = jnp.concatenate(
        [col2.reshape(p2), diag_idx,
         jnp.zeros((pad_f,), jnp.int32)]).reshape(rows_f, 128)
    exF = jnp.concatenate(
        [ex2.reshape(p2),
         jnp.zeros((n + pad_f,), jnp.float32)]).reshape(rows_f, 128)
    widF = jnp.concatenate(
        [wid2.reshape(p2),
         jnp.full((n + pad_f,), -1, jnp.int32)]).reshape(rows_f, 128)

    bmp2 = 200 if n % 200 == 0 else n
    dense0 = _zero_dense(n, bmp2)
    pw_ref = jax.new_ref(dense0.reshape(n * n))
    vscF = _sc_scatter_vals(cellF, rowF, colF, exF, widF, Zt, dvt, pw_ref)
    pw_adj = pw_ref[...].reshape(n, n)

    vexp = jnp.broadcast_to(
        vscF[:rows_p].reshape(p2)[:, None], (p2, d)).astype(jnp.float32)
    pe = _sc_spmm(vexp, row2, col2, gnn)

    logp = _head(gnn, pe[0, :n], pe[1, :n], dvt, W1, b1, W2, b2, 1.0, bm)
    return (logp, pw_adj)


# speedup vs baseline: 1.0049x; 1.0049x over previous
import functools
import jax
import jax.numpy as jnp
from jax import lax
from jax.experimental import pallas as pl
from jax.experimental.pallas import tpu as pltpu
from jax.experimental.pallas import tpu_sc as plsc


_GATHER_G = 8
_GATHER_LANES = 128


def _gcd(a, b):
    while b:
        a, b = b, a % b
    return a


def _sc_gather(table, idx_pad2, groups_per_worker):
    v, d = table.shape
    r = idx_pad2.shape[0]
    nw = 32
    rows_buf = _GATHER_G * _GATHER_LANES

    mesh = plsc.VectorSubcoreMesh(core_axis_name="c", subcore_axis_name="s")

    @functools.partial(
        pl.kernel,
        out_type=jax.ShapeDtypeStruct((r * _GATHER_LANES, d), jnp.float32),
        mesh=mesh,
        compiler_params=pltpu.CompilerParams(use_tc_tiling_on_sc=False),
        scratch_types=[
            pltpu.VMEM((_GATHER_G, _GATHER_LANES), jnp.int32),
            pltpu.VMEM((rows_buf, d), jnp.float32),
            pltpu.SemaphoreType.DMA,
        ],
    )
    def k(table_hbm, idx_hbm, out_hbm, idx_v, rows_v, sem):
        nc = 2
        wid = lax.axis_index("s") * nc + lax.axis_index("c")
        base = wid * (groups_per_worker * _GATHER_G)

        def body(g, carry):
            row0 = base + g * _GATHER_G
            pltpu.sync_copy(idx_hbm.at[pl.ds(row0, _GATHER_G)], idx_v)
            copies = []
            for j in range(_GATHER_G):
                copies.append(pltpu.async_copy(
                    table_hbm.at[idx_v.at[j]],
                    rows_v.at[pl.ds(j * _GATHER_LANES, _GATHER_LANES)],
                    sem))
            for c in copies:
                c.wait()
            pltpu.sync_copy(rows_v,
                            out_hbm.at[pl.ds(row0 * _GATHER_LANES, rows_buf)])
            return carry

        lax.fori_loop(0, groups_per_worker, body, None)

    return k(table, idx_pad2)


def _enc_body(x_ref, w_ref, o_ref):
    o_ref[...] = jnp.maximum(x_ref[...] @ w_ref[...], 0.0)


def _encode(features, W_pw, bm):
    n, k = features.shape
    d = W_pw.shape[1]
    return pl.pallas_call(
        _enc_body,
        grid=(n // bm,),
        in_specs=[
            pl.BlockSpec((bm, k), lambda i: (i, 0)),
            pl.BlockSpec((k, d), lambda i: (0, 0)),
        ],
        out_specs=pl.BlockSpec((bm, d), lambda i: (i, 0)),
        out_shape=jax.ShapeDtypeStruct((n, d), jnp.float32),
    )(features, W_pw)


def _prop_body(a_ref, x_ref, o_ref):
    o_ref[...] = a_ref[...] @ x_ref[...]


def _prop2_body(a_ref, x_ref, e_ref, t_ref, o_ref):
    o_ref[...] = (a_ref[...] @ x_ref[...] + e_ref[...] + t_ref[...]) * (1.0 / 3.0)


def _propagate(adj, emb0, bm):
    n, d = emb0.shape
    t1 = pl.pallas_call(
        _prop_body,
        grid=(n // bm,),
        in_specs=[
            pl.BlockSpec((bm, n), lambda i: (i, 0)),
            pl.BlockSpec((n, d), lambda i: (0, 0)),
        ],
        out_specs=pl.BlockSpec((bm, d), lambda i: (i, 0)),
        out_shape=jax.ShapeDtypeStruct((n, d), jnp.float32),
    )(adj, emb0)
    gnn = pl.pallas_call(
        _prop2_body,
        grid=(n // bm,),
        in_specs=[
            pl.BlockSpec((bm, n), lambda i: (i, 0)),
            pl.BlockSpec((n, d), lambda i: (0, 0)),
            pl.BlockSpec((bm, d), lambda i: (i, 0)),
            pl.BlockSpec((bm, d), lambda i: (i, 0)),
        ],
        out_specs=pl.BlockSpec((bm, d), lambda i: (i, 0)),
        out_shape=jax.ShapeDtypeStruct((n, d), jnp.float32),
    )(adj, t1, emb0, t1)
    return gnn


def _lstm_body(pe_ref, len_ref, wih_ref, whh_ref, b_ref, wo_ref, bo_ref,
               o_ref, *, nl, h_dim):
    x = pe_ref[...]
    bp = x.shape[0]
    b = b_ref[...]
    wcat = jnp.concatenate([wih_ref[...], whh_ref[...]], axis=0)
    idx = jnp.clip(len_ref[...] - 1, 0, nl - 1)
    h = jnp.zeros((bp, h_dim), jnp.float32)
    c = jnp.zeros((bp, h_dim), jnp.float32)
    h_last = jnp.zeros((bp, h_dim), jnp.float32)
    d = x.shape[1] // nl
    for l in range(nl):
        x_t = x[:, l * d:(l + 1) * d]
        z = jnp.concatenate([x_t, h], axis=1) @ wcat + b
        i_g = jax.nn.sigmoid(z[:, :h_dim])
        f_g = jax.nn.sigmoid(z[:, h_dim:2 * h_dim])
        g_g = jnp.tanh(z[:, 2 * h_dim:3 * h_dim])
        o_g = jax.nn.sigmoid(z[:, 3 * h_dim:])
        c = f_g * c + i_g * g_g
        h = o_g * jnp.tanh(c)
        h_last = jnp.where(idx == l, h, h_last)
    pw = jax.nn.sigmoid(h_last @ wo_ref[...] + bo_ref[0, 0])
    o_ref[...] = pw


def _lstm_pw(path_emb, lengths, W_ih, W_hh, b_ih, b_hh, w_out, b_out, bp):
    p, ld = path_emb.shape
    h_dim = W_hh.shape[1]
    nl = ld // (W_ih.shape[1])
    wih = W_ih.T
    whh = W_hh.T
    b = (b_ih + b_hh).reshape(1, -1)
    wo = w_out.reshape(-1, 1)
    bo = b_out.reshape(1, 1)
    lengths2 = lengths.reshape(p, 1)
    pw2 = pl.pallas_call(
        functools.partial(_lstm_body, nl=nl, h_dim=h_dim),
        grid=(p // bp,),
        in_specs=[
            pl.BlockSpec((bp, ld), lambda i: (i, 0)),
            pl.BlockSpec((bp, 1), lambda i: (i, 0)),
            pl.BlockSpec(wih.shape, lambda i: (0, 0)),
            pl.BlockSpec(whh.shape, lambda i: (0, 0)),
            pl.BlockSpec(b.shape, lambda i: (0, 0)),
            pl.BlockSpec(wo.shape, lambda i: (0, 0)),
            pl.BlockSpec(bo.shape, lambda i: (0, 0)),
        ],
        out_specs=pl.BlockSpec((bp, 1), lambda i: (i, 0)),
        out_shape=jax.ShapeDtypeStruct((p, 1), jnp.float32),
    )(path_emb, lengths2, wih, whh, b, wo, bo)
    return pw2.reshape(p)


_SC_PARAMS = pltpu.CompilerParams(use_tc_tiling_on_sc=False)
_NW = 32


def _sc_mesh():
    return plsc.VectorSubcoreMesh(core_axis_name="c", subcore_axis_name="s")


def _iota16():
    return lax.iota(jnp.int32, 16)


def _sc_scatter_ids(cell2, nn_cells):
    r = cell2.shape[0]
    wr = r // _NW

    @functools.partial(
        pl.kernel,
        out_type=jax.ShapeDtypeStruct((nn_cells,), jnp.int32),
        mesh=_sc_mesh(),
        compiler_params=_SC_PARAMS,
        scratch_types=[
            pltpu.VMEM((wr, 128), jnp.int32),
            pltpu.VMEM((20, 128), jnp.int32),
            pltpu.SemaphoreType.DMA,
        ],
    )
    def k(cell_hbm, out_hbm, cidx_v, vals_v, sem):
        w = lax.axis_index("s") * 2 + lax.axis_index("c")
        base = w * wr
        pltpu.sync_copy(cell_hbm.at[pl.ds(base, wr)], cidx_v)

        gsz = 20 if wr % 20 == 0 else 8

        def grp(g, carry):
            for j in range(gsz):
                rowbase = (base + g * gsz + j) * 128
                for q in range(8):
                    vals_v[j, pl.ds(q * 16, 16)] = rowbase + q * 16 + _iota16()
            cs = []
            for j in range(gsz):
                cs.append(pltpu.async_copy(
                    vals_v.at[j], out_hbm.at[cidx_v.at[g * gsz + j]], sem))
            for c in cs:
                c.wait()
            return carry

        lax.fori_loop(0, wr // gsz, grp, 0)

    return k(cell2)


def _sc_winner_extra(cell2, pw2, cellbuf):
    r = cell2.shape[0]
    wr = r // _NW
    p2 = r * 128
    sl = p2 // 16

    @functools.partial(
        pl.kernel,
        out_type=(jax.ShapeDtypeStruct((r, 128), jnp.int32),
                  jax.ShapeDtypeStruct((2, p2), jnp.float32)),
        mesh=_sc_mesh(),
        compiler_params=_SC_PARAMS,
        scratch_types=[
            pltpu.VMEM((wr, 128), jnp.int32),
            pltpu.VMEM((wr, 128), jnp.int32),
            pltpu.VMEM((wr, 128), jnp.float32),
            pltpu.VMEM((wr, 128), jnp.float32),
            pltpu.VMEM_SHARED((p2,), jnp.float32),
            pltpu.SemaphoreType.DMA,
        ],
    )
    def k(cell_hbm, pw_hbm, cb_hbm, z_hbm, wid_out, extra_out,
          cidx_v, wid_v, pw_v, lv_v, spart, sem):
        c = lax.axis_index("c")
        s = lax.axis_index("s")
        w = s * 2 + c
        pltpu.sync_copy(z_hbm.at[pl.ds(s * sl, sl)],
                        spart.at[pl.ds(s * sl, sl)])
        plsc.subcore_barrier()
        base = w * wr
        pltpu.sync_copy(cell_hbm.at[pl.ds(base, wr)], cidx_v)
        pltpu.sync_copy(pw_hbm.at[pl.ds(base, wr)], pw_v)

        gsz = 20 if wr % 20 == 0 else 8

        def ggrp(g, carry):
            cs = []
            for j in range(gsz):
                row = g * gsz + j
                cs.append(pltpu.async_copy(
                    cb_hbm.at[cidx_v.at[row]], wid_v.at[row], sem))
            for cpy in cs:
                cpy.wait()
            return carry

        lax.fori_loop(0, wr // gsz, ggrp, 0)

        def comp(j, carry):
            rowbase = (base + j) * 128
            for q in range(8):
                idv = rowbase + q * 16 + _iota16()
                wv = wid_v[j, pl.ds(q * 16, 16)]
                pv = pw_v[j, pl.ds(q * 16, 16)]
                lv_v[j, pl.ds(q * 16, 16)] = jnp.where(
                    wv == idv, jnp.zeros((16,), jnp.float32), pv)
            return carry

        lax.fori_loop(0, wr, comp, 0)

        def sgrp(j, carry):
            pltpu.sync_copy(lv_v.at[j], spart.at[wid_v.at[j]], add=True)
            return carry

        lax.fori_loop(0, wr, sgrp, 0)
        plsc.subcore_barrier()
        pltpu.sync_copy(spart.at[pl.ds(s * sl, sl)],
                        extra_out.at[c, pl.ds(s * sl, sl)])
        pltpu.sync_copy(wid_v, wid_out.at[pl.ds(base, wr)])

    return k(cell2, pw2, cellbuf, jnp.zeros((p2,), jnp.float32))


def _sc_stats(wid2, row2, col2, combined, n):
    r = wid2.shape[0]
    wr = r // _NW
    na = ((n + 127) // 128) * 128
    sl = na // 16

    @functools.partial(
        pl.kernel,
        out_type=(jax.ShapeDtypeStruct((r, 128), jnp.float32),
                  jax.ShapeDtypeStruct((2, na), jnp.float32),
                  jax.ShapeDtypeStruct((2, na), jnp.float32)),
        mesh=_sc_mesh(),
        compiler_params=_SC_PARAMS,
        scratch_types=[
            pltpu.VMEM((wr, 128), jnp.int32),
            pltpu.VMEM((wr, 128), jnp.int32),
            pltpu.VMEM((wr, 128), jnp.int32),
            pltpu.VMEM((wr, 128), jnp.float32),
            pltpu.VMEM((wr, 128), jnp.float32),
            pltpu.VMEM((wr, 128), jnp.float32),
            pltpu.VMEM_SHARED((na,), jnp.float32),
            pltpu.VMEM_SHARED((na,), jnp.float32),
            pltpu.VMEM_SHARED((r * 128,), jnp.float32),
            pltpu.SemaphoreType.DMA,
        ],
    )
    def k(wid_hbm, row_hbm, col_hbm, comb_hbm, z_hbm,
          ex_out, zsum_out, dsum_out,
          wid_v, row_v, col_v, t_v, zv_v, dv_v, zpart, dpart, spcomb, sem):
        c = lax.axis_index("c")
        s = lax.axis_index("s")
        w = s * 2 + c
        sl2 = (r * 128) // 16
        pltpu.sync_copy(z_hbm.at[pl.ds(s * sl, sl)],
                        zpart.at[pl.ds(s * sl, sl)])
        pltpu.sync_copy(z_hbm.at[pl.ds(s * sl, sl)],
                        dpart.at[pl.ds(s * sl, sl)])
        pltpu.sync_copy(comb_hbm.at[pl.ds(s * sl2, sl2)],
                        spcomb.at[pl.ds(s * sl2, sl2)])
        plsc.subcore_barrier()
        base = w * wr
        pltpu.sync_copy(wid_hbm.at[pl.ds(base, wr)], wid_v)
        pltpu.sync_copy(row_hbm.at[pl.ds(base, wr)], row_v)
        pltpu.sync_copy(col_hbm.at[pl.ds(base, wr)], col_v)

        def ggrp(g, carry):
            cs = []
            for j in range(8):
                row = g * 8 + j
                cs.append(pltpu.async_copy(
                    spcomb.at[wid_v.at[row]], t_v.at[row], sem))
            for cpy in cs:
                cpy.wait()
            return carry

        lax.fori_loop(0, wr // 8, ggrp, 0)

        def comp(j, carry):
            rowbase = (base + j) * 128
            for q in range(8):
                sli = pl.ds(q * 16, 16)
                idv = rowbase + q * 16 + _iota16()
                wv = wid_v[j, sli]
                rv = row_v[j, sli]
                cv = col_v[j, sli]
                tv = t_v[j, sli]
                ex = jnp.exp(tv)
                winner = wv == idv
                offd = rv != cv
                zero = jnp.zeros((16,), jnp.float32)
                exw = jnp.where(winner, ex, zero)
                tw = jnp.where(winner, tv, zero)
                zv_v[j, sli] = jnp.where(offd, exw, zero)
                dv_v[j, sli] = jnp.where(offd, zero, tw)
                t_v[j, sli] = ex
            return carry

        lax.fori_loop(0, wr, comp, 0)

        def sgrp(j, carry):
            pltpu.sync_copy(zv_v.at[j], zpart.at[row_v.at[j]], add=True)
            pltpu.sync_copy(dv_v.at[j], dpart.at[row_v.at[j]], add=True)
            return carry

        lax.fori_loop(0, wr, sgrp, 0)
        plsc.subcore_barrier()
        pltpu.sync_copy(zpart.at[pl.ds(s * sl, sl)],
                        zsum_out.at[c, pl.ds(s * sl, sl)])
        pltpu.sync_copy(dpart.at[pl.ds(s * sl, sl)],
                        dsum_out.at[c, pl.ds(s * sl, sl)])
        pltpu.sync_copy(t_v, ex_out.at[pl.ds(base, wr)])

    return k(wid2, row2, col2, combined, jnp.zeros((na,), jnp.float32))


def _sc_scatter_vals(cellF, rowF, colF, exF, widF, Z, dv, pw_ref):
    r = cellF.shape[0]
    wr = r // _NW
    ng = wr // 8
    rem = wr - ng * 8

    n = Z.shape[0]

    @functools.partial(
        pl.kernel,
        out_type=jax.ShapeDtypeStruct((r, 128), jnp.float32),
        mesh=_sc_mesh(),
        compiler_params=_SC_PARAMS,
        scratch_types=[
            pltpu.VMEM((wr, 128), jnp.int32),
            pltpu.VMEM((wr, 128), jnp.int32),
            pltpu.VMEM((wr, 128), jnp.int32),
            pltpu.VMEM((wr, 128), jnp.int32),
            pltpu.VMEM((wr, 128), jnp.float32),
            pltpu.VMEM_SHARED((n,), jnp.float32),
            pltpu.VMEM_SHARED((n,), jnp.float32),
            pltpu.VMEM((wr, 128), jnp.float32),
            pltpu.VMEM((wr, 128), jnp.float32),
            pltpu.VMEM((wr, 128), jnp.float32),
            pltpu.VMEM((wr, 128), jnp.float32),
            pltpu.SemaphoreType.DMA,
        ],
    )
    def k(cell_hbm, row_hbm, col_hbm, ex_hbm, wid_hbm, zt_hbm, dvt_hbm,
          pw_adj_ref, vsc_out,
          cidx_v, row_v, col_v, wid_v, ex_v, zt_s, dvt_s, zr_v, dvr_v,
          v_v, vsc_v, sem):
        s = lax.axis_index("s")
        w = s * 2 + lax.axis_index("c")
        base = w * wr

        @pl.when(s == 0)
        def _():
            pltpu.sync_copy(zt_hbm, zt_s)
            pltpu.sync_copy(dvt_hbm, dvt_s)

        pltpu.sync_copy(cell_hbm.at[pl.ds(base, wr)], cidx_v)
        pltpu.sync_copy(row_hbm.at[pl.ds(base, wr)], row_v)
        pltpu.sync_copy(col_hbm.at[pl.ds(base, wr)], col_v)
        pltpu.sync_copy(ex_hbm.at[pl.ds(base, wr)], ex_v)
        pltpu.sync_copy(wid_hbm.at[pl.ds(base, wr)], wid_v)
        plsc.subcore_barrier()

        def ggrp(g, carry):
            cs = []
            for j in range(8):
                rw = g * 8 + j
                cs.append(pltpu.async_copy(
                    zt_s.at[row_v.at[rw]], zr_v.at[rw], sem))
                cs.append(pltpu.async_copy(
                    dvt_s.at[row_v.at[rw]], dvr_v.at[rw], sem))
            for cpy in cs:
                cpy.wait()
            return carry

        lax.fori_loop(0, wr // 8, ggrp, 0)

        def comp(j, carry):
            rowbase = (base + j) * 128
            for q in range(8):
                sli = pl.ds(q * 16, 16)
                idv = rowbase + q * 16 + _iota16()
                rv = row_v[j, sli]
                cv = col_v[j, sli]
                wv = wid_v[j, sli]
                ex = ex_v[j, sli]
                diag = rv == cv
                zero = jnp.zeros((16,), jnp.float32)
                v = jnp.where(diag, dvr_v[j, sli], ex / zr_v[j, sli])
                v_v[j, sli] = v
                vw = jnp.where(wv == idv, v, zero)
                vsc_v[j, sli] = jnp.where(diag, zero, vw)
            return carry

        lax.fori_loop(0, wr, comp, 0)

        def sgrp(g, carry):
            cs = []
            for j in range(16):
                rw = g * 16 + j
                cs.append(pltpu.async_copy(
                    v_v.at[rw], pw_adj_ref.at[cidx_v.at[rw]], sem))
            for cpy in cs:
                cpy.wait()
            return carry

        lax.fori_loop(0, wr // 16, sgrp, 0)
        for j in range(wr - (wr // 16) * 16):
            rw = (wr // 16) * 16 + j
            pltpu.async_copy(
                v_v.at[rw], pw_adj_ref.at[cidx_v.at[rw]], sem).wait()
        pltpu.sync_copy(vsc_v, vsc_out.at[pl.ds(base, wr)])

    return k(cellF, rowF, colF, exF, widF, Z, dv, pw_ref)


def _sc_spmm(vexp, row2, col2, gnn):
    r = row2.shape[0]
    wr = r // _NW
    n, d = gnn.shape
    na = ((n + 127) // 128) * 128
    sl = na // 16

    @functools.partial(
        pl.kernel,
        out_type=jax.ShapeDtypeStruct((2, na, d), jnp.float32),
        mesh=_sc_mesh(),
        compiler_params=_SC_PARAMS,
        scratch_types=[
            pltpu.VMEM((wr, 128), jnp.int32),
            pltpu.VMEM((wr, 128), jnp.int32),
            pltpu.VMEM((128, 64), jnp.float32),
            pltpu.VMEM((128, 64), jnp.float32),
            pltpu.VMEM_SHARED((na, 64), jnp.float32),
            pltpu.SemaphoreType.DMA,
        ],
    )
    def k(vexp_hbm, row_hbm, col_hbm, gnn_hbm, z_hbm,
          pe_out, row_v, col_v, rows_v, vex_v, pepart, sem):
        c = lax.axis_index("c")
        s = lax.axis_index("s")
        w = s * 2 + c
        pltpu.sync_copy(z_hbm.at[pl.ds(s * sl, sl)],
                        pepart.at[pl.ds(s * sl, sl)])
        plsc.subcore_barrier()
        base = w * wr
        pltpu.sync_copy(row_hbm.at[pl.ds(base, wr)], row_v)
        pltpu.sync_copy(col_hbm.at[pl.ds(base, wr)], col_v)

        def body(j, carry):
            cg = pltpu.async_copy(gnn_hbm.at[col_v.at[j]], rows_v, sem)
            cv = pltpu.async_copy(
                vexp_hbm.at[pl.ds((base + j) * 128, 128)], vex_v, sem)
            cg.wait()
            cv.wait()
            for kk in range(128):
                for qq in range(4):
                    sli = pl.ds(qq * 16, 16)
                    rows_v[kk, sli] = rows_v[kk, sli] * vex_v[kk, sli]
            pltpu.sync_copy(rows_v, pepart.at[row_v.at[j]], add=True)
            return carry

        lax.fori_loop(0, wr, body, 0)
        plsc.subcore_barrier()
        pltpu.sync_copy(pepart.at[pl.ds(s * sl, sl)],
                        pe_out.at[c, pl.ds(s * sl, sl)])

    return k(vexp, row2, col2, gnn, jnp.zeros((na, d), jnp.float32))


def _comb_body(pw_ref, e0_ref, e1_ref, o_ref):
    o_ref[...] = pw_ref[...] + e0_ref[...] + e1_ref[...]


def _combine(pw2, e0, e1):
    r = pw2.shape[0]
    return pl.pallas_call(
        _comb_body,
        grid=(1,),
        in_specs=[pl.BlockSpec((r, 128), lambda i: (0, 0))] * 3,
        out_specs=pl.BlockSpec((r, 128), lambda i: (0, 0)),
        out_shape=jax.ShapeDtypeStruct((r, 128), jnp.float32),
    )(pw2, e0, e1)


def _zstats_body(z0_ref, z1_ref, d0_ref, d1_ref, z_ref, dv_ref):
    de = jnp.exp(1.0 + d0_ref[...] + d1_ref[...])
    zt = z0_ref[...] + z1_ref[...] + de
    z_ref[...] = zt
    dv_ref[...] = de / zt


def _zstats(zsum, dsum, n):
    return pl.pallas_call(
        _zstats_body,
        grid=(1,),
        in_specs=[pl.BlockSpec((1, n), lambda i: (0, 0))] * 4,
        out_specs=[pl.BlockSpec((1, n), lambda i: (0, 0))] * 2,
        out_shape=[jax.ShapeDtypeStruct((1, n), jnp.float32)] * 2,
    )(zsum[0:1], zsum[1:2], dsum[0:1], dsum[1:2])


def _zfill_body(o_ref):
    o_ref[...] = jnp.zeros_like(o_ref)


def _zero_dense(n, bm):
    return pl.pallas_call(
        _zfill_body,
        grid=(n // bm,),
        in_specs=[],
        out_specs=pl.BlockSpec((bm, n), lambda i: (i, 0)),
        out_shape=jax.ShapeDtypeStruct((n, n), jnp.float32),
    )()


def _smax_body(a_ref, g_ref, o_ref, e_ref):
    a = a_ref[...]
    aw = jnp.where(a > 0.0, a, jnp.float32(-9e15))
    m = jnp.max(aw, axis=1, keepdims=True)
    ex = jnp.exp(aw - m)
    s = jnp.sum(ex, axis=1, keepdims=True)
    p = ex / s
    o_ref[...] = p
    e_ref[...] = p @ g_ref[...]


def _softmax_spmm(A, gnn, bm):
    n = A.shape[0]
    d = gnn.shape[1]
    return pl.pallas_call(
        _smax_body,
        grid=(n // bm,),
        in_specs=[
            pl.BlockSpec((bm, n), lambda i: (i, 0)),
            pl.BlockSpec((n, d), lambda i: (0, 0)),
        ],
        out_specs=[
            pl.BlockSpec((bm, n), lambda i: (i, 0)),
            pl.BlockSpec((bm, d), lambda i: (i, 0)),
        ],
        out_shape=[
            jax.ShapeDtypeStruct((n, n), jnp.float32),
            jax.ShapeDtypeStruct((n, d), jnp.float32),
        ],
    )(A, gnn)


def _head_body(g_ref, p0_ref, p1_ref, dvv_ref, w1_ref, b1_ref, w2_ref,
               b2_ref, o_ref, *, lam):
    g = g_ref[...]
    pe = p0_ref[...] + p1_ref[...] + dvv_ref[...] * g
    e = jnp.concatenate([g, lam * pe], axis=1)
    h = jnp.maximum(e @ w1_ref[...] + b1_ref[...], 0.0)
    lg = h @ w2_ref[...] + b2_ref[...]
    m = jnp.max(lg, axis=1, keepdims=True)
    lse = m + jnp.log(jnp.sum(jnp.exp(lg - m), axis=1, keepdims=True))
    o_ref[...] = lg - lse


def _head(gnn, pe0, pe1, dvv, W1, b1, W2, b2, lam, bm):
    n, d = gnn.shape
    nh = W1.shape[1]
    nc = W2.shape[1]
    return pl.pallas_call(
        functools.partial(_head_body, lam=lam),
        grid=(n // bm,),
        in_specs=[
            pl.BlockSpec((bm, d), lambda i: (i, 0)),
            pl.BlockSpec((bm, d), lambda i: (i, 0)),
            pl.BlockSpec((bm, d), lambda i: (i, 0)),
            pl.BlockSpec((bm, 1), lambda i: (i, 0)),
            pl.BlockSpec(W1.shape, lambda i: (0, 0)),
            pl.BlockSpec((1, nh), lambda i: (0, 0)),
            pl.BlockSpec(W2.shape, lambda i: (0, 0)),
            pl.BlockSpec((1, nc), lambda i: (0, 0)),
        ],
        out_specs=pl.BlockSpec((bm, nc), lambda i: (i, 0)),
        out_shape=jax.ShapeDtypeStruct((n, nc), jnp.float32),
    )(gnn, pe0, pe1, dvv.reshape(n, 1), W1, b1.reshape(1, -1),
      W2, b2.reshape(1, -1))


def kernel(features, adj, pairs, sub_paths, sub_path_length, W_pw, W_ih, W_hh,
           b_ih, b_hh, w_out, b_out, W1, b1, W2, b2):
    n = features.shape[0]
    d = W_pw.shape[1]
    p, l = sub_paths.shape

    bm_enc = 2000 if n % 2000 == 0 else n
    emb0 = _encode(features, W_pw, bm_enc)

    bm = 1000 if n % 1000 == 0 else n
    bmp = 200 if n % 200 == 0 else n
    gnn = _propagate(adj, emb0, bmp)

    flat_idx = sub_paths.reshape(-1).astype(jnp.int32)
    unit = 32 * _GATHER_G * _GATHER_LANES
    flat_unit = (unit // _gcd(unit, l)) * l
    npad = ((p * l + flat_unit - 1) // flat_unit) * flat_unit
    idx_pad = jnp.pad(flat_idx, (0, npad - p * l))
    p_pad = npad // l
    lengths = jnp.pad(sub_path_length.astype(jnp.int32), (0, p_pad - p))
    nunits = npad // flat_unit
    nchunks = 1
    for cand in (5, 2):
        if nunits % cand == 0:
            nchunks = cand
            break
    cflat = npad // nchunks
    cpaths = p_pad // nchunks
    bp = 2048 if cpaths % 2048 == 0 else (2000 if cpaths % 2000 == 0 else cpaths)
    pw_parts = []
    for ci in range(nchunks):
        idx2 = idx_pad[ci * cflat:(ci + 1) * cflat].reshape(-1, _GATHER_LANES)
        rows = _sc_gather(gnn, idx2, cflat // unit)
        pe = rows.reshape(cpaths, l * d)
        ln = lengths[ci * cpaths:(ci + 1) * cpaths]
        pw_parts.append(_lstm_pw(pe, ln, W_ih, W_hh, b_ih, b_hh,
                                 w_out, b_out, bp))
    pw = jnp.concatenate(pw_parts)[:p]

    rowi = pairs[:, 0].astype(jnp.int32)
    coli = pairs[:, 1].astype(jnp.int32)
    celli = rowi * n + coli
    rows_p = ((p + 128 * 256 - 1) // (128 * 256)) * 256
    p2 = rows_p * 128
    pad_p = p2 - p
    cell2 = jnp.pad(celli, (0, pad_p)).reshape(rows_p, 128)
    row2 = jnp.pad(rowi, (0, pad_p)).reshape(rows_p, 128)
    col2 = jnp.pad(coli, (0, pad_p)).reshape(rows_p, 128)
    pw2 = jnp.pad(pw, (0, pad_p)).reshape(rows_p, 128)

    cellbuf = _sc_scatter_ids(cell2, n * n)
    wid2, extra = _sc_winner_extra(cell2, pw2, cellbuf)
    comb2 = _combine(pw2, extra[0].reshape(rows_p, 128),
                     extra[1].reshape(rows_p, 128))
    ex2, zsum, dsum = _sc_stats(wid2, row2, col2, comb2.reshape(p2), n)
    Zt, dvt = _zstats(zsum[:, :n], dsum[:, :n], n)
    Zt = Zt.reshape(n)
    dvt = dvt.reshape(n)

    rows_f = ((p2 + n + 128 * 256 - 1) // (128 * 256)) * 256
    pf = rows_f * 128
    pad_f = pf - p2 - n
    diag_idx = jnp.arange(n, dtype=jnp.int32)
    cellF = jnp.concatenate(
        [cell2.reshape(p2), diag_idx * (n + 1),
         jnp.zeros((pad_f,), jnp.int32)]).reshape(rows_f, 128)
    rowF = jnp.concatenate(
        [row2.reshape(p2), diag_idx,
         jnp.zeros((pad_f,), jnp.int32)]).reshape(rows_f, 128)
    colF = jnp.concatenate(
        [col2.reshape(p2), diag_idx,
         jnp.zeros((pad_f,), jnp.int32)]).reshape(rows_f, 128)
    exF = jnp.concatenate(
        [ex2.reshape(p2),
         jnp.zeros((n + pad_f,), jnp.float32)]).reshape(rows_f, 128)
    widF = jnp.concatenate(
        [wid2.reshape(p2),
         jnp.full((n + pad_f,), -1, jnp.int32)]).reshape(rows_f, 128)

    bmp2 = 200 if n % 200 == 0 else n
    dense0 = _zero_dense(n, bmp2)
    pw_ref = jax.new_ref(dense0.reshape(n * n))
    vscF = _sc_scatter_vals(cellF, rowF, colF, exF, widF, Zt, dvt, pw_ref)
    pw_adj = pw_ref[...].reshape(n, n)

    vexp = jnp.broadcast_to(
        vscF[:rows_p].reshape(p2)[:, None], (p2, d)).astype(jnp.float32)
    pe = _sc_spmm(vexp, row2, col2, gnn)

    logp = _head(gnn, pe[0, :n], pe[1, :n], dvt, W1, b1, W2, b2, 1.0, bm)
    return (logp, pw_adj)


# speedup vs baseline: 2.6565x; 2.6436x over previous
import functools
import jax
import jax.numpy as jnp
from jax import lax
from jax.experimental import pallas as pl
from jax.experimental.pallas import tpu as pltpu
from jax.experimental.pallas import tpu_sc as plsc


_GATHER_G = 8
_GATHER_LANES = 128


def _gcd(a, b):
    while b:
        a, b = b, a % b
    return a


def _sc_gather(table, idx_pad2, groups_per_worker):
    v, d = table.shape
    r = idx_pad2.shape[0]
    nw = 32
    rows_buf = _GATHER_G * _GATHER_LANES

    mesh = plsc.VectorSubcoreMesh(core_axis_name="c", subcore_axis_name="s")

    @functools.partial(
        pl.kernel,
        out_type=jax.ShapeDtypeStruct((r * _GATHER_LANES, d), jnp.float32),
        mesh=mesh,
        compiler_params=pltpu.CompilerParams(use_tc_tiling_on_sc=False),
        scratch_types=[
            pltpu.VMEM((_GATHER_G, _GATHER_LANES), jnp.int32),
            pltpu.VMEM((rows_buf, d), jnp.float32),
            pltpu.SemaphoreType.DMA,
        ],
    )
    def k(table_hbm, idx_hbm, out_hbm, idx_v, rows_v, sem):
        nc = 2
        wid = lax.axis_index("s") * nc + lax.axis_index("c")
        base = wid * (groups_per_worker * _GATHER_G)

        def body(g, carry):
            row0 = base + g * _GATHER_G
            pltpu.sync_copy(idx_hbm.at[pl.ds(row0, _GATHER_G)], idx_v)
            copies = []
            for j in range(_GATHER_G):
                copies.append(pltpu.async_copy(
                    table_hbm.at[idx_v.at[j]],
                    rows_v.at[pl.ds(j * _GATHER_LANES, _GATHER_LANES)],
                    sem))
            for c in copies:
                c.wait()
            pltpu.sync_copy(rows_v,
                            out_hbm.at[pl.ds(row0 * _GATHER_LANES, rows_buf)])
            return carry

        lax.fori_loop(0, groups_per_worker, body, None)

    return k(table, idx_pad2)


def _enc_body(x_ref, w_ref, o_ref):
    o_ref[...] = jnp.maximum(x_ref[...] @ w_ref[...], 0.0)


def _encode(features, W_pw, bm):
    n, k = features.shape
    d = W_pw.shape[1]
    return pl.pallas_call(
        _enc_body,
        grid=(n // bm,),
        in_specs=[
            pl.BlockSpec((bm, k), lambda i: (i, 0)),
            pl.BlockSpec((k, d), lambda i: (0, 0)),
        ],
        out_specs=pl.BlockSpec((bm, d), lambda i: (i, 0)),
        out_shape=jax.ShapeDtypeStruct((n, d), jnp.float32),
    )(features, W_pw)


def _prop_body(a_ref, x_ref, o_ref):
    o_ref[...] = a_ref[...] @ x_ref[...]


def _prop2_body(a_ref, x_ref, e_ref, t_ref, o_ref):
    o_ref[...] = (a_ref[...] @ x_ref[...] + e_ref[...] + t_ref[...]) * (1.0 / 3.0)


def _propagate(adj, emb0, bm):
    n, d = emb0.shape
    t1 = pl.pallas_call(
        _prop_body,
        grid=(n // bm,),
        in_specs=[
            pl.BlockSpec((bm, n), lambda i: (i, 0)),
            pl.BlockSpec((n, d), lambda i: (0, 0)),
        ],
        out_specs=pl.BlockSpec((bm, d), lambda i: (i, 0)),
        out_shape=jax.ShapeDtypeStruct((n, d), jnp.float32),
    )(adj, emb0)
    gnn = pl.pallas_call(
        _prop2_body,
        grid=(n // bm,),
        in_specs=[
            pl.BlockSpec((bm, n), lambda i: (i, 0)),
            pl.BlockSpec((n, d), lambda i: (0, 0)),
            pl.BlockSpec((bm, d), lambda i: (i, 0)),
            pl.BlockSpec((bm, d), lambda i: (i, 0)),
        ],
        out_specs=pl.BlockSpec((bm, d), lambda i: (i, 0)),
        out_shape=jax.ShapeDtypeStruct((n, d), jnp.float32),
    )(adj, t1, emb0, t1)
    return gnn


def _lstm_body(pe_ref, len_ref, wih_ref, whh_ref, b_ref, wo_ref, bo_ref,
               o_ref, *, nl, h_dim):
    x = pe_ref[...]
    bp = x.shape[0]
    b = b_ref[...]
    wcat = jnp.concatenate([wih_ref[...], whh_ref[...]], axis=0)
    idx = jnp.clip(len_ref[...] - 1, 0, nl - 1)
    h = jnp.zeros((bp, h_dim), jnp.float32)
    c = jnp.zeros((bp, h_dim), jnp.float32)
    h_last = jnp.zeros((bp, h_dim), jnp.float32)
    d = x.shape[1] // nl
    for l in range(nl):
        x_t = x[:, l * d:(l + 1) * d]
        z = jnp.concatenate([x_t, h], axis=1) @ wcat + b
        i_g = jax.nn.sigmoid(z[:, :h_dim])
        f_g = jax.nn.sigmoid(z[:, h_dim:2 * h_dim])
        g_g = jnp.tanh(z[:, 2 * h_dim:3 * h_dim])
        o_g = jax.nn.sigmoid(z[:, 3 * h_dim:])
        c = f_g * c + i_g * g_g
        h = o_g * jnp.tanh(c)
        h_last = jnp.where(idx == l, h, h_last)
    pw = jax.nn.sigmoid(h_last @ wo_ref[...] + bo_ref[0, 0])
    o_ref[...] = pw


def _lstm_pw(path_emb, lengths, W_ih, W_hh, b_ih, b_hh, w_out, b_out, bp):
    p, ld = path_emb.shape
    h_dim = W_hh.shape[1]
    nl = ld // (W_ih.shape[1])
    wih = W_ih.T
    whh = W_hh.T
    b = (b_ih + b_hh).reshape(1, -1)
    wo = w_out.reshape(-1, 1)
    bo = b_out.reshape(1, 1)
    lengths2 = lengths.reshape(p, 1)
    pw2 = pl.pallas_call(
        functools.partial(_lstm_body, nl=nl, h_dim=h_dim),
        grid=(p // bp,),
        in_specs=[
            pl.BlockSpec((bp, ld), lambda i: (i, 0)),
            pl.BlockSpec((bp, 1), lambda i: (i, 0)),
            pl.BlockSpec(wih.shape, lambda i: (0, 0)),
            pl.BlockSpec(whh.shape, lambda i: (0, 0)),
            pl.BlockSpec(b.shape, lambda i: (0, 0)),
            pl.BlockSpec(wo.shape, lambda i: (0, 0)),
            pl.BlockSpec(bo.shape, lambda i: (0, 0)),
        ],
        out_specs=pl.BlockSpec((bp, 1), lambda i: (i, 0)),
        out_shape=jax.ShapeDtypeStruct((p, 1), jnp.float32),
    )(path_emb, lengths2, wih, whh, b, wo, bo)
    return pw2.reshape(p)


def _smax_body(a_ref, g_ref, o_ref, e_ref):
    a = a_ref[...]
    aw = jnp.where(a > 0.0, a, jnp.float32(-9e15))
    m = jnp.max(aw, axis=1, keepdims=True)
    ex = jnp.exp(aw - m)
    s = jnp.sum(ex, axis=1, keepdims=True)
    p = ex / s
    o_ref[...] = p
    e_ref[...] = p @ g_ref[...]


def _softmax_spmm(A, gnn, bm):
    n = A.shape[0]
    d = gnn.shape[1]
    return pl.pallas_call(
        _smax_body,
        grid=(n // bm,),
        in_specs=[
            pl.BlockSpec((bm, n), lambda i: (i, 0)),
            pl.BlockSpec((n, d), lambda i: (0, 0)),
        ],
        out_specs=[
            pl.BlockSpec((bm, n), lambda i: (i, 0)),
            pl.BlockSpec((bm, d), lambda i: (i, 0)),
        ],
        out_shape=[
            jax.ShapeDtypeStruct((n, n), jnp.float32),
            jax.ShapeDtypeStruct((n, d), jnp.float32),
        ],
    )(A, gnn)


def _head_body(g_ref, pe_ref, w1_ref, b1_ref, w2_ref, b2_ref, o_ref, *, lam):
    e = jnp.concatenate([g_ref[...], lam * pe_ref[...]], axis=1)
    h = jnp.maximum(e @ w1_ref[...] + b1_ref[...], 0.0)
    lg = h @ w2_ref[...] + b2_ref[...]
    m = jnp.max(lg, axis=1, keepdims=True)
    lse = m + jnp.log(jnp.sum(jnp.exp(lg - m), axis=1, keepdims=True))
    o_ref[...] = lg - lse


def _head(gnn, pw_emd, W1, b1, W2, b2, lam, bm):
    n, d = gnn.shape
    nh = W1.shape[1]
    nc = W2.shape[1]
    return pl.pallas_call(
        functools.partial(_head_body, lam=lam),
        grid=(n // bm,),
        in_specs=[
            pl.BlockSpec((bm, d), lambda i: (i, 0)),
            pl.BlockSpec((bm, d), lambda i: (i, 0)),
            pl.BlockSpec(W1.shape, lambda i: (0, 0)),
            pl.BlockSpec((1, nh), lambda i: (0, 0)),
            pl.BlockSpec(W2.shape, lambda i: (0, 0)),
            pl.BlockSpec((1, nc), lambda i: (0, 0)),
        ],
        out_specs=pl.BlockSpec((bm, nc), lambda i: (i, 0)),
        out_shape=jax.ShapeDtypeStruct((n, nc), jnp.float32),
    )(gnn, pw_emd, W1, b1.reshape(1, -1), W2, b2.reshape(1, -1))


def kernel(features, adj, pairs, sub_paths, sub_path_length, W_pw, W_ih, W_hh,
           b_ih, b_hh, w_out, b_out, W1, b1, W2, b2):
    n = features.shape[0]
    d = W_pw.shape[1]
    p, l = sub_paths.shape

    bm_enc = 2000 if n % 2000 == 0 else n
    emb0 = _encode(features, W_pw, bm_enc)

    bm = 1000 if n % 1000 == 0 else n
    bmp = 200 if n % 200 == 0 else n
    gnn = _propagate(adj, emb0, bmp)

    flat_idx = sub_paths.reshape(-1).astype(jnp.int32)
    unit = 32 * _GATHER_G * _GATHER_LANES
    flat_unit = (unit // _gcd(unit, l)) * l
    npad = ((p * l + flat_unit - 1) // flat_unit) * flat_unit
    idx_pad = jnp.pad(flat_idx, (0, npad - p * l))
    p_pad = npad // l
    lengths = jnp.pad(sub_path_length.astype(jnp.int32), (0, p_pad - p))
    nunits = npad // flat_unit
    nchunks = 1
    for cand in (5, 2):
        if nunits % cand == 0:
            nchunks = cand
            break
    cflat = npad // nchunks
    cpaths = p_pad // nchunks
    bp = 2048 if cpaths % 2048 == 0 else (2000 if cpaths % 2000 == 0 else cpaths)
    pw_parts = []
    for ci in range(nchunks):
        idx2 = idx_pad[ci * cflat:(ci + 1) * cflat].reshape(-1, _GATHER_LANES)
        rows = _sc_gather(gnn, idx2, cflat // unit)
        pe = rows.reshape(cpaths, l * d)
        ln = lengths[ci * cpaths:(ci + 1) * cpaths]
        pw_parts.append(_lstm_pw(pe, ln, W_ih, W_hh, b_ih, b_hh,
                                 w_out, b_out, bp))
    pw = jnp.concatenate(pw_parts)[:p]

    A = jnp.zeros((n, n), jnp.float32).at[pairs[:, 0], pairs[:, 1]].add(pw)
    diag = jnp.arange(n)
    A = A.at[diag, diag].add(1.0)

    bs = 200 if n % 200 == 0 else n
    pw_adj, pw_emd = _softmax_spmm(A, gnn, bs)

    logp = _head(gnn, pw_emd, W1, b1, W2, b2, 1.0, bm)
    return (logp, pw_adj)
